# baseline (speedup 1.0000x reference)
import functools

import jax
import jax.numpy as jnp
from jax import lax
from jax.experimental import pallas as pl
from jax.experimental.pallas import tpu as pltpu
from jax.experimental.pallas import tpu_sc as plsc

E = 64
NG = 8
GSZ = E // NG
TOPKG = 3
K = 8
BLOCK_T = 2048

NC = 2
NS = 16
NW = NC * NS
L = 16

_SORT8 = [(0, 1), (2, 3), (4, 5), (6, 7),
          (0, 2), (1, 3), (4, 6), (5, 7),
          (1, 2), (5, 6),
          (0, 4), (1, 5), (2, 6), (3, 7),
          (2, 4), (3, 5),
          (1, 2), (3, 4), (5, 6)]
_BITONIC8 = [(0, 4), (1, 5), (2, 6), (3, 7),
             (0, 2), (1, 3), (4, 6), (5, 7),
             (0, 1), (2, 3), (4, 5), (6, 7)]


def _score_kernel(x_ref, w_ref, s_ref):
    logits = lax.dot_general(
        w_ref[...], x_ref[...],
        dimension_numbers=(((1,), (1,)), ((), ())),
        preferred_element_type=jnp.float32,
    )
    m = jnp.max(logits, axis=0, keepdims=True)
    unnorm = jnp.exp(logits - m)
    s_ref[...] = unnorm / jnp.sum(unnorm, axis=0, keepdims=True)


def _cmpx(k, i, a, b):
    c = k[a] >= k[b]
    ka, kb = jnp.where(c, k[a], k[b]), jnp.where(c, k[b], k[a])
    ia, ib = jnp.where(c, i[a], i[b]), jnp.where(c, i[b], i[a])
    k[a], k[b], i[a], i[b] = ka, kb, ia, ib


def _sort8(k, i):
    for a, b in _SORT8:
        _cmpx(k, i, a, b)


def _merge8(ak, ai, bk, bi):
    wk, wi = [], []
    for j in range(8):
        c = ak[j] >= bk[7 - j]
        wk.append(jnp.where(c, ak[j], bk[7 - j]))
        wi.append(jnp.where(c, ai[j], bi[7 - j]))
    for a, b in _BITONIC8:
        _cmpx(wk, wi, a, b)
    return wk, wi


def _make_route_kernel(n_tok):
    tpw = n_tok // NW
    ntile = tpw // L
    mesh = plsc.VectorSubcoreMesh(core_axis_name="c", subcore_axis_name="s")

    @functools.partial(
        pl.kernel, mesh=mesh,
        out_type=[
            jax.ShapeDtypeStruct((K, n_tok), jnp.int32),
            jax.ShapeDtypeStruct((K, n_tok), jnp.float32),
        ],
        scratch_types=[
            pltpu.VMEM((E, tpw), jnp.float32),
            pltpu.VMEM((K, tpw), jnp.int32),
            pltpu.VMEM((K, tpw), jnp.float32),
        ],
        compiler_params=pltpu.CompilerParams(needs_layout_passes=False),
    )
    def route(scores_hbm, idx_hbm, wgt_hbm, sbuf, ibuf, wbuf):
        wid = lax.axis_index("s") * NC + lax.axis_index("c")
        base = wid * tpw
        pltpu.sync_copy(scores_hbm.at[:, pl.ds(base, tpw)], sbuf)

        lane = lax.broadcasted_iota(jnp.int32, (L,), 0)
        gids = [jnp.full((L,), g, jnp.int32) for g in range(NG)]

        @plsc.parallel_loop(0, ntile, step=1, unroll=4)
        def body(tile):
            tok = tile * L + lane
            gk = []
            for g in range(NG):
                m = sbuf[g * GSZ, pl.ds(tile * L, L)]
                for o in range(1, GSZ):
                    m = jnp.maximum(m, sbuf[g * GSZ + o, pl.ds(tile * L, L)])
                gk.append(m)
            gi = list(gids)
            _sort8(gk, gi)
            ck, ci = [], []
            for slot in range(TOPKG):
                erow0 = gi[slot] * GSZ
                for o in range(GSZ):
                    eid = erow0 + o
                    ck.append(plsc.load_gather(sbuf, [eid, tok]))
                    ci.append(eid)
            k0, i0 = ck[0:8], ci[0:8]
            k1, i1 = ck[8:16], ci[8:16]
            k2, i2 = ck[16:24], ci[16:24]
            _sort8(k0, i0)
            _sort8(k1, i1)
            _sort8(k2, i2)
            mk, mi = _merge8(k0, i0, k1, i1)
            fk, fi = _merge8(mk, mi, k2, i2)
            total = fk[0]
            for r in range(1, K):
                total = total + fk[r]
            total = total + 1e-20
            for r in range(K):
                ibuf[r, pl.ds(tile * L, L)] = fi[r]
                wbuf[r, pl.ds(tile * L, L)] = fk[r] / total
        pltpu.sync_copy(ibuf, idx_hbm.at[:, pl.ds(base, tpw)])
        pltpu.sync_copy(wbuf, wgt_hbm.at[:, pl.ds(base, tpw)])

    return route


@jax.jit
def kernel(hidden_states, weight):
    bsz, seq, h = hidden_states.shape
    x = hidden_states.reshape(-1, h)
    n_tok = x.shape[0]
    scores = pl.pallas_call(
        _score_kernel,
        grid=(n_tok // BLOCK_T,),
        in_specs=[
            pl.BlockSpec((BLOCK_T, h), lambda i: (i, 0)),
            pl.BlockSpec((E, h), lambda i: (0, 0)),
        ],
        out_specs=pl.BlockSpec((E, BLOCK_T), lambda i: (0, i)),
        out_shape=jax.ShapeDtypeStruct((E, n_tok), jnp.float32),
        compiler_params=pltpu.CompilerParams(
            dimension_semantics=("arbitrary",),
        ),
    )(x, weight)
    idx_t, wgt_t = _make_route_kernel(n_tok)(scores)
    return idx_t.T, wgt_t.T, None


# speedup vs baseline: 1.0054x; 1.0054x over previous
import functools

import jax
import jax.numpy as jnp
from jax import lax
from jax.experimental import pallas as pl
from jax.experimental.pallas import tpu as pltpu
from jax.experimental.pallas import tpu_sc as plsc

E = 64
NG = 8
GSZ = E // NG
TOPKG = 3
K = 8
BLOCK_T = 2048

NC = 2
NS = 16
NW = NC * NS
L = 16

_SORT8 = [(0, 1), (2, 3), (4, 5), (6, 7),
          (0, 2), (1, 3), (4, 6), (5, 7),
          (1, 2), (5, 6),
          (0, 4), (1, 5), (2, 6), (3, 7),
          (2, 4), (3, 5),
          (1, 2), (3, 4), (5, 6)]
_BITONIC8 = [(0, 4), (1, 5), (2, 6), (3, 7),
             (0, 2), (1, 3), (4, 6), (5, 7),
             (0, 1), (2, 3), (4, 5), (6, 7)]


def _score_kernel(x_ref, w_ref, s_ref, g_ref):
    logits = lax.dot_general(
        w_ref[...], x_ref[...],
        dimension_numbers=(((1,), (1,)), ((), ())),
        preferred_element_type=jnp.float32,
    )
    m = jnp.max(logits, axis=0, keepdims=True)
    unnorm = jnp.exp(logits - m)
    scores = unnorm / jnp.sum(unnorm, axis=0, keepdims=True)
    s_ref[...] = scores
    T = scores.shape[1]
    gmax = jnp.concatenate(
        [jnp.max(scores[g * GSZ:(g + 1) * GSZ, :], axis=0, keepdims=True)
         for g in range(NG)], axis=0)
    giota = lax.broadcasted_iota(jnp.int32, (NG, T), 0)
    avail = gmax
    rows = []
    for _ in range(TOPKG):
        gm = jnp.max(avail, axis=0, keepdims=True)
        gsel = jnp.min(jnp.where(avail == gm, giota, NG), axis=0,
                       keepdims=True)
        rows.append(gsel)
        avail = jnp.where(giota == gsel, -1.0, avail)
    g_ref[...] = jnp.concatenate(rows + [rows[0]], axis=0)


def _cmpx(k, i, a, b):
    c = k[a] >= k[b]
    ka, kb = jnp.where(c, k[a], k[b]), jnp.where(c, k[b], k[a])
    ia, ib = jnp.where(c, i[a], i[b]), jnp.where(c, i[b], i[a])
    k[a], k[b], i[a], i[b] = ka, kb, ia, ib


def _sort8(k, i):
    for a, b in _SORT8:
        _cmpx(k, i, a, b)


def _merge8(ak, ai, bk, bi):
    wk, wi = [], []
    for j in range(8):
        c = ak[j] >= bk[7 - j]
        wk.append(jnp.where(c, ak[j], bk[7 - j]))
        wi.append(jnp.where(c, ai[j], bi[7 - j]))
    for a, b in _BITONIC8:
        _cmpx(wk, wi, a, b)
    return wk, wi


def _make_route_kernel(n_tok):
    tpw = n_tok // NW
    ntile = tpw // L
    mesh = plsc.VectorSubcoreMesh(core_axis_name="c", subcore_axis_name="s")

    @functools.partial(
        pl.kernel, mesh=mesh,
        out_type=[
            jax.ShapeDtypeStruct((K, n_tok), jnp.int32),
            jax.ShapeDtypeStruct((K, n_tok), jnp.float32),
        ],
        scratch_types=[
            pltpu.VMEM((E, tpw), jnp.float32),
            pltpu.VMEM((TOPKG + 1, tpw), jnp.int32),
            pltpu.VMEM((K, tpw), jnp.int32),
            pltpu.VMEM((K, tpw), jnp.float32),
        ],
        compiler_params=pltpu.CompilerParams(needs_layout_passes=False),
    )
    def route(scores_hbm, gidx_hbm, idx_hbm, wgt_hbm, sbuf, gbuf, ibuf, wbuf):
        wid = lax.axis_index("s") * NC + lax.axis_index("c")
        base = wid * tpw
        pltpu.sync_copy(scores_hbm.at[:, pl.ds(base, tpw)], sbuf)
        pltpu.sync_copy(gidx_hbm.at[:, pl.ds(base, tpw)], gbuf)

        lane = lax.broadcasted_iota(jnp.int32, (L,), 0)

        @plsc.parallel_loop(0, ntile, step=1, unroll=2)
        def body(tile):
            tok = tile * L + lane
            ck, ci = [], []
            for slot in range(TOPKG):
                erow0 = gbuf[slot, pl.ds(tile * L, L)] * GSZ
                for o in range(GSZ):
                    eid = erow0 + o
                    ck.append(plsc.load_gather(sbuf, [eid, tok]))
                    ci.append(eid)
            k0, i0 = ck[0:8], ci[0:8]
            k1, i1 = ck[8:16], ci[8:16]
            k2, i2 = ck[16:24], ci[16:24]
            _sort8(k0, i0)
            _sort8(k1, i1)
            _sort8(k2, i2)
            mk, mi = _merge8(k0, i0, k1, i1)
            fk, fi = _merge8(mk, mi, k2, i2)
            total = fk[0]
            for r in range(1, K):
                total = total + fk[r]
            total = total + 1e-20
            for r in range(K):
                ibuf[r, pl.ds(tile * L, L)] = fi[r]
                wbuf[r, pl.ds(tile * L, L)] = fk[r] / total
        pltpu.sync_copy(ibuf, idx_hbm.at[:, pl.ds(base, tpw)])
        pltpu.sync_copy(wbuf, wgt_hbm.at[:, pl.ds(base, tpw)])

    return route


@jax.jit
def kernel(hidden_states, weight):
    bsz, seq, h = hidden_states.shape
    x = hidden_states.reshape(-1, h)
    n_tok = x.shape[0]
    outs = pl.pallas_call(
        _score_kernel,
        grid=(n_tok // BLOCK_T,),
        in_specs=[
            pl.BlockSpec((BLOCK_T, h), lambda i: (i, 0)),
            pl.BlockSpec((E, h), lambda i: (0, 0)),
        ],
        out_specs=[
            pl.BlockSpec((E, BLOCK_T), lambda i: (0, i)),
            pl.BlockSpec((TOPKG + 1, BLOCK_T), lambda i: (0, i)),
        ],
        out_shape=[
            jax.ShapeDtypeStruct((E, n_tok), jnp.float32),
            jax.ShapeDtypeStruct((TOPKG + 1, n_tok), jnp.int32),
        ],
        compiler_params=pltpu.CompilerParams(
            dimension_semantics=("arbitrary",),
        ),
    )(x, weight)
    scores, gidx = outs
    idx_t, wgt_t = _make_route_kernel(n_tok)(scores, gidx)
    return idx_t.T, wgt_t.T, None
